# baseline (speedup 1.0000x reference)
import functools

import jax
import jax.numpy as jnp
import numpy as np
from jax import lax
from jax.experimental import pallas as pl
from jax.experimental.pallas import tpu as pltpu
from jax.experimental.pallas import tpu_sc as plsc


def _sc_gather(emb, idx_pad):
    g_pad = idx_pad.shape[0]
    d = emb.shape[1]
    info = plsc.get_sparse_core_info()
    nc, ns = info.num_cores, info.num_subcores
    nw = nc * ns
    b_per_w = g_pad // nw

    mesh = plsc.VectorSubcoreMesh(core_axis_name="c", subcore_axis_name="s")

    @functools.partial(
        pl.kernel,
        mesh=mesh,
        out_type=jax.ShapeDtypeStruct((g_pad, d), jnp.float32),
        scratch_types=[
            pltpu.VMEM((b_per_w,), jnp.int32),
            pltpu.VMEM((b_per_w, d), jnp.float32),
            pltpu.SemaphoreType.DMA,
        ],
    )
    def gather_kernel(table_hbm, idx_hbm, out_hbm, idx_v, rows_v, sem):
        wid = lax.axis_index("s") * nc + lax.axis_index("c")
        base = wid * b_per_w
        pltpu.sync_copy(idx_hbm.at[pl.ds(base, b_per_w)], idx_v)
        pltpu.async_copy(table_hbm.at[idx_v], rows_v, sem).wait()
        pltpu.sync_copy(rows_v, out_hbm.at[pl.ds(base, b_per_w)])

    return gather_kernel(emb, idx_pad)


def _tc_body(xt_ref, f_ref, s_ref, b_ref, o_ref):
    xtb = xt_ref[...].astype(jnp.bfloat16)
    fb = f_ref[...].astype(jnp.bfloat16)
    h = lax.dot_general(
        xtb, fb, (((0,), (0,)), ((), ())), preferred_element_type=jnp.float32
    )
    h = 0.5 * h * (1.0 + lax.erf(h * np.float32(1.0 / np.sqrt(2.0))))
    mu = jnp.mean(h, axis=-1, keepdims=True)
    m2 = jnp.mean(h * h, axis=-1, keepdims=True)
    var = m2 - mu * mu
    inv = lax.rsqrt(var + np.float32(1e-5))
    o_ref[...] = (h - mu) * (inv * s_ref[...]) + b_ref[...]


def _tc_main(x, feature, ln_scale, ln_bias, bm=1024):
    batch, g = x.shape
    d = feature.shape[1]
    xt = x.T
    return pl.pallas_call(
        _tc_body,
        grid=(batch // bm,),
        in_specs=[
            pl.BlockSpec((g, bm), lambda i: (0, i)),
            pl.BlockSpec((g, d), lambda i: (0, 0)),
            pl.BlockSpec((1, d), lambda i: (0, 0)),
            pl.BlockSpec((1, d), lambda i: (0, 0)),
        ],
        out_specs=pl.BlockSpec((bm, d), lambda i: (i, 0)),
        out_shape=jax.ShapeDtypeStruct((batch, d), jnp.float32),
        compiler_params=pltpu.CompilerParams(
            dimension_semantics=("parallel",),
        ),
    )(xt, feature, ln_scale.reshape(1, d), ln_bias.reshape(1, d))


def kernel(x, input_gene_idx, emb, ln_scale, ln_bias):
    g, d = emb.shape
    g_pad = 1024
    idx_pad = jnp.pad(input_gene_idx, (0, g_pad - g))
    feature_pad = _sc_gather(emb, idx_pad)
    out = _tc_main(x, feature_pad, ln_scale, ln_bias)
    return (out, input_gene_idx)


# speedup vs baseline: 1.0485x; 1.0485x over previous
import functools

import jax
import jax.numpy as jnp
import numpy as np
from jax import lax
from jax.experimental import pallas as pl
from jax.experimental.pallas import tpu as pltpu
from jax.experimental.pallas import tpu_sc as plsc


def _sc_gather(emb, idx_pad):
    g_pad = idx_pad.shape[0]
    d = emb.shape[1]
    info = plsc.get_sparse_core_info()
    nc, ns = info.num_cores, info.num_subcores
    nw = nc * ns
    b_per_w = g_pad // nw

    mesh = plsc.VectorSubcoreMesh(core_axis_name="c", subcore_axis_name="s")

    @functools.partial(
        pl.kernel,
        mesh=mesh,
        out_type=jax.ShapeDtypeStruct((g_pad, d), jnp.float32),
        scratch_types=[
            pltpu.VMEM((b_per_w,), jnp.int32),
            pltpu.VMEM((b_per_w, d), jnp.float32),
            pltpu.SemaphoreType.DMA,
        ],
    )
    def gather_kernel(table_hbm, idx_hbm, out_hbm, idx_v, rows_v, sem):
        wid = lax.axis_index("s") * nc + lax.axis_index("c")
        base = wid * b_per_w
        pltpu.sync_copy(idx_hbm.at[pl.ds(base, b_per_w)], idx_v)
        pltpu.async_copy(table_hbm.at[idx_v], rows_v, sem).wait()
        pltpu.sync_copy(rows_v, out_hbm.at[pl.ds(base, b_per_w)])

    return gather_kernel(emb, idx_pad)


def _tc_body(xt_ref, f_ref, s_ref, b_ref, o_ref):
    xtb = xt_ref[...].astype(jnp.bfloat16)
    fb = f_ref[...].astype(jnp.bfloat16)
    h = lax.dot_general(
        xtb, fb, (((0,), (0,)), ((), ())), preferred_element_type=jnp.float32
    )
    h = 0.5 * h * (1.0 + lax.erf(h * np.float32(1.0 / np.sqrt(2.0))))
    mu = jnp.mean(h, axis=-1, keepdims=True)
    m2 = jnp.mean(h * h, axis=-1, keepdims=True)
    var = m2 - mu * mu
    inv = lax.rsqrt(var + np.float32(1e-5))
    o_ref[...] = (h - mu) * (inv * s_ref[...]) + b_ref[...]


def _tc_main(x, feature, ln_scale, ln_bias, bm=2048):
    batch, g = x.shape
    d = feature.shape[1]
    xt = x.T
    return pl.pallas_call(
        _tc_body,
        grid=(batch // bm,),
        in_specs=[
            pl.BlockSpec((g, bm), lambda i: (0, i)),
            pl.BlockSpec((g, d), lambda i: (0, 0)),
            pl.BlockSpec((1, d), lambda i: (0, 0)),
            pl.BlockSpec((1, d), lambda i: (0, 0)),
        ],
        out_specs=pl.BlockSpec((bm, d), lambda i: (i, 0)),
        out_shape=jax.ShapeDtypeStruct((batch, d), jnp.float32),
        compiler_params=pltpu.CompilerParams(
            dimension_semantics=("parallel",),
        ),
    )(xt, feature, ln_scale.reshape(1, d), ln_bias.reshape(1, d))


def kernel(x, input_gene_idx, emb, ln_scale, ln_bias):
    g, d = emb.shape
    g_pad = 1024
    idx_pad = jnp.pad(input_gene_idx, (0, g_pad - g))
    feature_pad = _sc_gather(emb, idx_pad)
    out = _tc_main(x, feature_pad, ln_scale, ln_bias)
    return (out, input_gene_idx)


# speedup vs baseline: 1.0744x; 1.0247x over previous
import functools

import jax
import jax.numpy as jnp
import numpy as np
from jax import lax
from jax.experimental import pallas as pl
from jax.experimental.pallas import tpu as pltpu
from jax.experimental.pallas import tpu_sc as plsc


def _sc_gather(emb, idx):
    g, d = emb.shape
    info = plsc.get_sparse_core_info()
    nc, ns = info.num_cores, info.num_subcores
    nw = nc * ns
    b_per_w = 32
    assert (nw - 1) * b_per_w < g <= nw * b_per_w and (g - b_per_w) % 8 == 0

    mesh = plsc.VectorSubcoreMesh(core_axis_name="c", subcore_axis_name="s")

    @functools.partial(
        pl.kernel,
        mesh=mesh,
        out_type=(
            jax.ShapeDtypeStruct((g, d), jnp.float32),
            jax.ShapeDtypeStruct((g,), jnp.int32),
        ),
        scratch_types=[
            pltpu.VMEM((b_per_w,), jnp.int32),
            pltpu.VMEM((b_per_w, d), jnp.float32),
            pltpu.SemaphoreType.DMA,
        ],
    )
    def gather_kernel(table_hbm, idx_hbm, feat_hbm, idxo_hbm, idx_v, rows_v, sem):
        wid = lax.axis_index("s") * nc + lax.axis_index("c")
        base = jnp.minimum(wid * b_per_w, g - b_per_w)
        pltpu.sync_copy(idx_hbm.at[pl.ds(base, b_per_w)], idx_v)
        pltpu.async_copy(table_hbm.at[idx_v], rows_v, sem).wait()
        pltpu.sync_copy(rows_v, feat_hbm.at[pl.ds(base, b_per_w)])
        pltpu.sync_copy(idx_v, idxo_hbm.at[pl.ds(base, b_per_w)])

    return gather_kernel(emb, idx)


def _tc_body(xt_ref, f_ref, s_ref, b_ref, o_ref):
    xtb = xt_ref[...].astype(jnp.bfloat16)
    fb = f_ref[...].astype(jnp.bfloat16)
    h = lax.dot_general(
        xtb, fb, (((0,), (0,)), ((), ())), preferred_element_type=jnp.float32
    )
    h = 0.5 * h * (1.0 + lax.erf(h * np.float32(1.0 / np.sqrt(2.0))))
    mu = jnp.mean(h, axis=-1, keepdims=True)
    m2 = jnp.mean(h * h, axis=-1, keepdims=True)
    var = m2 - mu * mu
    inv = lax.rsqrt(var + np.float32(1e-5))
    o_ref[...] = (h - mu) * (inv * s_ref[...]) + b_ref[...]


def _tc_main(x, feature, ln_scale, ln_bias, bm=2048):
    batch, g = x.shape
    d = feature.shape[1]
    xt = x.T
    return pl.pallas_call(
        _tc_body,
        grid=(batch // bm,),
        in_specs=[
            pl.BlockSpec((g, bm), lambda i: (0, i)),
            pl.BlockSpec((g, d), lambda i: (0, 0)),
            pl.BlockSpec((1, d), lambda i: (0, 0)),
            pl.BlockSpec((1, d), lambda i: (0, 0)),
        ],
        out_specs=pl.BlockSpec((bm, d), lambda i: (i, 0)),
        out_shape=jax.ShapeDtypeStruct((batch, d), jnp.float32),
        compiler_params=pltpu.CompilerParams(
            dimension_semantics=("parallel",),
        ),
    )(xt, feature, ln_scale.reshape(1, d), ln_bias.reshape(1, d))


def kernel(x, input_gene_idx, emb, ln_scale, ln_bias):
    feature, idx_out = _sc_gather(emb, input_gene_idx)
    out = _tc_main(x, feature, ln_scale, ln_bias)
    return (out, idx_out)


# speedup vs baseline: 1.0753x; 1.0009x over previous
import functools

import jax
import jax.numpy as jnp
import numpy as np
from jax import lax
from jax.experimental import pallas as pl
from jax.experimental.pallas import tpu as pltpu
from jax.experimental.pallas import tpu_sc as plsc


def _sc_gather(emb, idx):
    g, d = emb.shape
    info = plsc.get_sparse_core_info()
    nc, ns = info.num_cores, info.num_subcores
    nc = 1
    nw = nc * ns
    b_per_w = 64
    assert (nw - 1) * b_per_w < g <= nw * b_per_w and (g - b_per_w) % 8 == 0

    mesh = plsc.VectorSubcoreMesh(core_axis_name="c", subcore_axis_name="s", num_cores=1)

    @functools.partial(
        pl.kernel,
        mesh=mesh,
        out_type=(
            jax.ShapeDtypeStruct((g, d), jnp.float32),
            jax.ShapeDtypeStruct((g,), jnp.int32),
        ),
        scratch_types=[
            pltpu.VMEM((b_per_w,), jnp.int32),
            pltpu.VMEM((b_per_w, d), jnp.float32),
            pltpu.SemaphoreType.DMA,
        ],
    )
    def gather_kernel(table_hbm, idx_hbm, feat_hbm, idxo_hbm, idx_v, rows_v, sem):
        wid = lax.axis_index("s") * nc + lax.axis_index("c")
        base = jnp.minimum(wid * b_per_w, g - b_per_w)
        pltpu.sync_copy(idx_hbm.at[pl.ds(base, b_per_w)], idx_v)
        pltpu.async_copy(table_hbm.at[idx_v], rows_v, sem).wait()
        pltpu.sync_copy(rows_v, feat_hbm.at[pl.ds(base, b_per_w)])
        pltpu.sync_copy(idx_v, idxo_hbm.at[pl.ds(base, b_per_w)])

    return gather_kernel(emb, idx)


def _tc_body(xt_ref, f_ref, s_ref, b_ref, o_ref):
    xtb = xt_ref[...].astype(jnp.bfloat16)
    fb = f_ref[...].astype(jnp.bfloat16)
    h = lax.dot_general(
        xtb, fb, (((0,), (0,)), ((), ())), preferred_element_type=jnp.float32
    )
    h = 0.5 * h * (1.0 + lax.erf(h * np.float32(1.0 / np.sqrt(2.0))))
    mu = jnp.mean(h, axis=-1, keepdims=True)
    m2 = jnp.mean(h * h, axis=-1, keepdims=True)
    var = m2 - mu * mu
    inv = lax.rsqrt(var + np.float32(1e-5))
    o_ref[...] = (h - mu) * (inv * s_ref[...]) + b_ref[...]


def _tc_main(x, feature, ln_scale, ln_bias, bm=2048):
    batch, g = x.shape
    d = feature.shape[1]
    xt = x.T
    return pl.pallas_call(
        _tc_body,
        grid=(batch // bm,),
        in_specs=[
            pl.BlockSpec((g, bm), lambda i: (0, i)),
            pl.BlockSpec((g, d), lambda i: (0, 0)),
            pl.BlockSpec((1, d), lambda i: (0, 0)),
            pl.BlockSpec((1, d), lambda i: (0, 0)),
        ],
        out_specs=pl.BlockSpec((bm, d), lambda i: (i, 0)),
        out_shape=jax.ShapeDtypeStruct((batch, d), jnp.float32),
        compiler_params=pltpu.CompilerParams(
            dimension_semantics=("parallel",),
        ),
    )(xt, feature, ln_scale.reshape(1, d), ln_bias.reshape(1, d))


def kernel(x, input_gene_idx, emb, ln_scale, ln_bias):
    feature, idx_out = _sc_gather(emb, input_gene_idx)
    out = _tc_main(x, feature, ln_scale, ln_bias)
    return (out, idx_out)
